# baseline (speedup 1.0000x reference)
import numpy as np
import jax
import jax.numpy as jnp
from jax import lax
from jax.experimental import pallas as pl
from jax.experimental.pallas import tpu as pltpu
from jax.experimental.pallas import tpu_sc as plsc

_ANCHORS = np.array([
    [[0.28, 0.22], [0.38, 0.48], [0.90, 0.78]],
    [[0.07, 0.15], [0.15, 0.11], [0.14, 0.29]],
    [[0.02, 0.03], [0.04, 0.07], [0.08, 0.06]],
], dtype=np.float32)
_S_LIST = [13, 26, 52]
_BATCH = 32
_NC, _NS = 2, 16
_NW = _NC * _NS

_N_CELLS = [_BATCH * 3 * s * s for s in _S_LIST]
_CH = [512, 2032, 8112]
_SS = [s * s for s in _S_LIST]

_anch_tab = np.zeros(32, np.float32)
for _i in range(3):
    for _k in range(3):
        _anch_tab[_i * 8 + 2 * _k] = 1.0 / (_ANCHORS[_i, _k, 0] * _S_LIST[_i])
        _anch_tab[_i * 8 + 2 * _k + 1] = 1.0 / (_ANCHORS[_i, _k, 1] * _S_LIST[_i])

_LN2 = 0.6931471805599453
_SQRT2 = 1.4142135623730951


def _softlog(x):
    b = plsc.bitcast(x, jnp.int32)
    e = lax.shift_right_logical(b, 23) - 127
    m = plsc.bitcast((b & 0x007FFFFF) | 0x3F800000, jnp.float32)
    big = m >= _SQRT2
    m = jnp.where(big, m * 0.5, m)
    e = e + jnp.where(big, 1, 0)
    t = (m - 1.0) / (m + 1.0)
    t2 = t * t
    p = 1.0 + t2 * (1.0 / 3.0 + t2 * (0.2 + t2 * (1.0 / 7.0 + t2 * (1.0 / 9.0))))
    return e.astype(jnp.float32) * _LN2 + 2.0 * t * p


def _do_scale(i, pred_ref, tgt_ref, out_ref, tgt_buf, idx_cell, idx_ebuf,
              gath_buf, anch_v, stage, sem, wid):
    n_cells, ch, ss = _N_CELLS[i], _CH[i], _SS[i]
    lane = lax.iota(jnp.int32, 16)
    base_owned = wid * ch
    base_read = jnp.minimum(base_owned, n_cells - ch)
    skip = base_owned - base_read

    pltpu.sync_copy(tgt_ref.at[pl.ds(base_read * 6, ch * 6)],
                    tgt_buf.at[pl.ds(0, ch * 6)])

    def scan_body(j, m_vec):
        lidx = j * 16 + lane
        t0 = plsc.load_gather(tgt_buf, [lidx * 6])
        msk = (t0 == 1.0) & (lidx >= skip)
        pos = m_vec + plsc.cumsum(msk.astype(jnp.int32)) - 1
        plsc.store_scatter(idx_cell, [pos], lidx, mask=msk)
        return m_vec + plsc.all_reduce_population_count(msk)

    m_vec = lax.fori_loop(0, ch // 16, scan_body, jnp.zeros(16, jnp.int32))
    m = jnp.max(m_vec)
    nch = (m + 31) // 32

    def chunk_body(ci, acc):
        cells = []
        for v in range(2):
            cpos = ci * 32 + v * 16 + lane
            vld = cpos < m
            cell = plsc.load_gather(idx_cell, [cpos])
            cell = jnp.where(vld, cell, 0)
            cells.append((cell, vld))
            fb = (base_read + cell) * 4
            for c in range(4):
                idx_ebuf[pl.ds(c * 32 + v * 16, 16)] = fb + c
        pltpu.async_copy(pred_ref.at[idx_ebuf], gath_buf, sem).wait()
        for v in range(2):
            cell, vld = cells[v]
            tb = cell * 6
            tx = plsc.load_gather(tgt_buf, [tb + 1])
            ty = plsc.load_gather(tgt_buf, [tb + 2])
            tw = plsc.load_gather(tgt_buf, [tb + 3])
            th = plsc.load_gather(tgt_buf, [tb + 4])
            px = gath_buf[pl.ds(0 * 32 + v * 16, 16)]
            py = gath_buf[pl.ds(1 * 32 + v * 16, 16)]
            pw = gath_buf[pl.ds(2 * 32 + v * 16, 16)]
            ph = gath_buf[pl.ds(3 * 32 + v * 16, 16)]
            k = ((base_read + cell) // ss) % 3
            iw = plsc.load_gather(anch_v, [i * 8 + 2 * k])
            ih = plsc.load_gather(anch_v, [i * 8 + 2 * k + 1])
            sx = 1.0 / (1.0 + jnp.exp(-px))
            sy = 1.0 / (1.0 + jnp.exp(-py))
            lw = _softlog(1e-16 + tw * iw)
            lh = _softlog(1e-16 + th * ih)
            dx, dy, dw, dh = sx - tx, sy - ty, pw - lw, ph - lh
            d = dx * dx + dy * dy + dw * dw + dh * dh
            acc = acc + jnp.where(vld, d, 0.0)
        return acc

    acc = lax.fori_loop(0, nch, chunk_body, jnp.zeros(16, jnp.float32))

    stage[...] = acc
    pltpu.sync_copy(stage, out_ref.at[wid, i])
    stage[...] = m_vec.astype(jnp.float32)
    pltpu.sync_copy(stage, out_ref.at[wid, 3 + i])


def _body(p0, p1, p2, t0, t1, t2, anch, out_ref, tgt_buf, idx_cell, idx_ebuf,
          gath_buf, anch_v, stage, sem):
    wid = lax.axis_index("s") * _NC + lax.axis_index("c")
    pltpu.sync_copy(anch, anch_v)
    preds = [p0, p1, p2]
    tgts = [t0, t1, t2]
    for i in range(3):
        _do_scale(i, preds[i], tgts[i], out_ref, tgt_buf, idx_cell, idx_ebuf,
                  gath_buf, anch_v, stage, sem, wid)


_sc_call = pl.kernel(
    _body,
    out_type=jax.ShapeDtypeStruct((_NW, 8, 16), jnp.float32),
    mesh=plsc.VectorSubcoreMesh(core_axis_name="c", subcore_axis_name="s",
                                num_cores=_NC, num_subcores=_NS),
    compiler_params=pltpu.CompilerParams(needs_layout_passes=False),
    scratch_types=[
        pltpu.VMEM((_CH[2] * 6,), jnp.float32),
        pltpu.VMEM((8192,), jnp.int32),
        pltpu.VMEM((128,), jnp.int32),
        pltpu.VMEM((128,), jnp.float32),
        pltpu.VMEM((32,), jnp.float32),
        pltpu.VMEM((16,), jnp.float32),
        pltpu.SemaphoreType.DMA,
    ],
)


def kernel(pred_0, pred_1, pred_2, target_0, target_1, target_2):
    p = [x[..., 1:5].reshape(-1) for x in (pred_0, pred_1, pred_2)]
    t = [x.reshape(-1) for x in (target_0, target_1, target_2)]
    anch = jnp.asarray(_anch_tab)
    parts = _sc_call(p[0], p[1], p[2], t[0], t[1], t[2], anch)
    s = parts[:, 0:3, :].sum(axis=(0, 2))
    cnt = parts[:, 3:6, 0].sum(axis=0)
    return (10.0 * s / jnp.maximum(4.0 * cnt, 1.0)).sum()
